# baseline (speedup 1.0000x reference)
import functools

import jax
import jax.numpy as jnp
from jax import lax
from jax.experimental import pallas as pl
from jax.experimental.pallas import tpu as pltpu
from jax.experimental.pallas import tpu_sc as plsc

_B = 16
_P = 1024
_D = 96
_L = 16
_TROWS = _B * _D
_NC = 2
_NS = 16
_NW = _NC * _NS
_RPW = _TROWS // _NW
_CH = 8
_NCH = _RPW // _CH
_NBUF = 6

_mesh = plsc.VectorSubcoreMesh(core_axis_name="c", subcore_axis_name="s")


@functools.partial(
    pl.kernel,
    mesh=_mesh,
    out_type=jax.ShapeDtypeStruct((_TROWS, _P), jnp.float32),
    scratch_types=[
        [pltpu.VMEM((_CH, _P), jnp.float32) for _ in range(_NBUF)],
        pltpu.VMEM((_D,), jnp.float32),
        pltpu.VMEM((_RPW, _L), jnp.float32),
        [pltpu.SemaphoreType.DMA for _ in range(_NBUF)],
        [pltpu.SemaphoreType.DMA for _ in range(_NBUF)],
    ],
    compiler_params=pltpu.CompilerParams(use_tc_tiling_on_sc=True),
)
def _encode(patch_hbm, pos_hbm, out_hbm, bufs, posv, splats, lsems, ssems):
    wid = lax.axis_index("s") * _NC + lax.axis_index("c")
    base = wid * _RPW
    dbase = (wid % 2) * _RPW

    loads = [None] * _NCH
    stores = [None] * _NCH
    for c in range(_NCH):
        loads[c] = pltpu.async_copy(
            patch_hbm.at[pl.ds(base + c * _CH, _CH)], bufs[c], lsems[c])

    pltpu.sync_copy(pos_hbm, posv)
    for g in range(_RPW // _L):
        vg = posv[pl.ds(dbase + g * _L, _L)]
        for k in range(_L):
            splats[g * _L + k, :] = lax.broadcast(vg[k], (_L,))

    for c in range(_NCH):
        b = c % _NBUF
        loads[c].wait()
        buf = bufs[b]

        def row(r, carry, buf=buf, c=c):
            pv = splats[c * _CH + r, :]
            for j in range(_P // _L):
                plsc.addupdate(buf.at[r, pl.ds(j * _L, _L)], pv)
            return carry

        lax.fori_loop(0, _CH, row, 0)
        stores[c] = pltpu.async_copy(
            buf, out_hbm.at[pl.ds(base + c * _CH, _CH)], ssems[b])

    for c in range(_NCH):
        stores[c].wait()


def kernel(patch, pos_table):
    pt = patch.transpose(0, 2, 1).reshape(_TROWS, _P)
    out = _encode(pt, pos_table[0])
    return out.reshape(_B, _D, _P).transpose(0, 2, 1)


# speedup vs baseline: 1.1455x; 1.1455x over previous
import functools

import jax
import jax.numpy as jnp
from jax import lax
from jax.experimental import pallas as pl
from jax.experimental.pallas import tpu as pltpu
from jax.experimental.pallas import tpu_sc as plsc

_B = 16
_P = 1024
_D = 96
_L = 16
_TROWS = _B * _D
_NC = 2
_NS = 16
_NW = _NC * _NS
_RPW = _TROWS // _NW
_CH = 8
_NCH = _RPW // _CH
_NBUF = 4

_mesh = plsc.VectorSubcoreMesh(core_axis_name="c", subcore_axis_name="s")


@functools.partial(
    pl.kernel,
    mesh=_mesh,
    out_type=jax.ShapeDtypeStruct((_TROWS, _P), jnp.float32),
    scratch_types=[
        [pltpu.VMEM((_CH, _P), jnp.float32) for _ in range(_NBUF)],
        pltpu.VMEM((_D,), jnp.float32),
        pltpu.VMEM((_RPW, _L), jnp.float32),
        [pltpu.SemaphoreType.DMA for _ in range(_NBUF)],
        [pltpu.SemaphoreType.DMA for _ in range(_NBUF)],
    ],
    compiler_params=pltpu.CompilerParams(use_tc_tiling_on_sc=True),
)
def _encode(patch_hbm, pos_hbm, out_hbm, bufs, posv, splats, lsems, ssems):
    wid = lax.axis_index("s") * _NC + lax.axis_index("c")
    base = wid * _RPW
    dbase = (wid % 2) * _RPW

    loads = [None] * _NCH
    stores = [None] * _NCH
    for c in range(2):
        loads[c] = pltpu.async_copy(
            patch_hbm.at[pl.ds(base + c * _CH, _CH)], bufs[c], lsems[c])

    pltpu.sync_copy(pos_hbm, posv)
    for g in range(_RPW // _L):
        vg = posv[pl.ds(dbase + g * _L, _L)]
        for k in range(_L):
            splats[g * _L + k, :] = lax.broadcast(vg[k], (_L,))

    for c in range(_NCH):
        b = c % _NBUF
        look = c + 2
        if look < _NCH:
            lb = look % _NBUF
            if look - _NBUF >= 0:
                stores[look - _NBUF].wait()
            loads[look] = pltpu.async_copy(
                patch_hbm.at[pl.ds(base + look * _CH, _CH)], bufs[lb], lsems[lb])
        loads[c].wait()
        buf = bufs[b]

        @functools.partial(plsc.parallel_loop, 0, _CH, unroll=2)
        def row(r, buf=buf, c=c):
            pv = splats[c * _CH + r, :]
            for j in range(_P // _L):
                buf[r, pl.ds(j * _L, _L)] += pv
        stores[c] = pltpu.async_copy(
            buf, out_hbm.at[pl.ds(base + c * _CH, _CH)], ssems[b])

    for c in range(_NCH - _NBUF, _NCH):
        stores[c].wait()


def kernel(patch, pos_table):
    pt = patch.transpose(0, 2, 1).reshape(_TROWS, _P)
    out = _encode(pt, pos_table[0])
    return out.reshape(_B, _D, _P).transpose(0, 2, 1)
